# Initial kernel scaffold; baseline (speedup 1.0000x reference)
import jax
import jax.numpy as jnp
from jax.experimental import pallas as pl


def kernel(nfeat, efeat, edge_index, W, b):
    raise NotImplementedError("write your pallas kernel here")


# speedup vs baseline: 6.9946x; 6.9946x over previous
import functools

import jax
import jax.numpy as jnp
from jax import lax
from jax.experimental import pallas as pl
from jax.experimental.pallas import tpu as pltpu
from jax.experimental.pallas import tpu_sc as plsc

N = 10000
E = 320000
D = 128
DE = 16

NC = 2
NS = 16
NW = NC * NS
EPW = E // NW
CHUNK = 40
NCHUNK = EPW // CHUNK
NROWC = N // CHUNK


def _zero_acc(acc_sh, iota_hbm, ia_v, z_v, lo, hi):
    @pl.loop(0, hi - lo)
    def _z(i):
        o = (lo + i) * CHUNK
        pltpu.sync_copy(iota_hbm.at[pl.ds(o, CHUNK)], ia_v)
        pltpu.sync_copy(z_v, acc_sh.at[ia_v])


def _read_acc(acc_sh, iota_hbm, out_hbm, ia_v, st_v, lo, hi, c, sem):
    @pl.loop(0, hi - lo)
    def _r(i):
        o = (lo + i) * CHUNK
        pltpu.sync_copy(iota_hbm.at[pl.ds(o, CHUNK)], ia_v)
        pltpu.async_copy(acc_sh.at[ia_v], st_v, sem).wait()
        pltpu.sync_copy(st_v, out_hbm.at[pl.ds(c * N + o, CHUNK)])


def _sc_body(nfeat_hbm, efeat_hbm, src_hbm, dst_hbm, z128_hbm, oh_hbm,
             iota_hbm, ps_hbm, ptg_hbm,
             acc_sh, src_v, dst_v, rows_v, big_v, ef_v, oh_v, ia_v, sem):
    c = lax.axis_index("c")
    s = lax.axis_index("s")
    wid = s * NC + c
    base = wid * EPW

    lo = (s * NROWC) // NS
    hi = ((s + 1) * NROWC) // NS

    pltpu.sync_copy(z128_hbm, rows_v)
    _zero_acc(acc_sh, iota_hbm, ia_v, rows_v, lo, hi)
    plsc.subcore_barrier()

    @pl.loop(0, NCHUNK)
    def _chunk(i):
        e0 = base + i * CHUNK
        pltpu.sync_copy(src_hbm.at[pl.ds(e0, CHUNK)], src_v)
        pltpu.sync_copy(dst_hbm.at[pl.ds(e0, CHUNK)], dst_v)
        pltpu.async_copy(nfeat_hbm.at[src_v], rows_v, sem).wait()
        pltpu.sync_copy(rows_v, acc_sh.at[dst_v], add=True)

    plsc.subcore_barrier()
    _read_acc(acc_sh, iota_hbm, ps_hbm, ia_v, rows_v, lo, hi, c, sem)
    plsc.subcore_barrier()

    pltpu.sync_copy(z128_hbm, big_v)
    _zero_acc(acc_sh, iota_hbm, ia_v, big_v, lo, hi)
    pltpu.sync_copy(oh_hbm, oh_v)
    for r in range(CHUNK):
        big_v[r, pl.ds(DE, 16)] = oh_v[r, :]
    plsc.subcore_barrier()

    @pl.loop(0, NCHUNK)
    def _chunk2(i):
        e0 = base + i * CHUNK
        pltpu.sync_copy(dst_hbm.at[pl.ds(e0, CHUNK)], dst_v)
        pltpu.sync_copy(efeat_hbm.at[pl.ds(e0, CHUNK)], ef_v)
        for r in range(CHUNK):
            big_v[r, pl.ds(0, DE)] = ef_v[r, :]
        pltpu.sync_copy(big_v, acc_sh.at[dst_v], add=True)

    plsc.subcore_barrier()
    _read_acc(acc_sh, iota_hbm, ptg_hbm, ia_v, big_v, lo, hi, c, sem)


@functools.cache
def _get_sc_call():
    return pl.kernel(
        _sc_body,
        out_type=[
            jax.ShapeDtypeStruct((2 * N, D), jnp.float32),
            jax.ShapeDtypeStruct((2 * N, D), jnp.float32),
        ],
        mesh=plsc.VectorSubcoreMesh(core_axis_name="c", subcore_axis_name="s",
                                    num_cores=NC, num_subcores=NS),
        scratch_types=[
            pltpu.VMEM_SHARED((N, D), jnp.float32),
            pltpu.VMEM((CHUNK,), jnp.int32),
            pltpu.VMEM((CHUNK,), jnp.int32),
            pltpu.VMEM((CHUNK, D), jnp.float32),
            pltpu.VMEM((CHUNK, D), jnp.float32),
            pltpu.VMEM((CHUNK, DE), jnp.float32),
            pltpu.VMEM((CHUNK, 16), jnp.float32),
            pltpu.VMEM((CHUNK,), jnp.int32),
            pltpu.SemaphoreType.DMA,
        ],
    )


ROWS = 1000
GRID = N // ROWS


def _tc_body(x_ref, s0_ref, s1_ref, p0_ref, p1_ref,
             wd_ref, ws_ref, we_ref, b_ref, o_ref):
    p = p0_ref[...] + p1_ref[...]
    deg = p[:, DE:DE + 1]
    acc = jnp.dot(x_ref[...], wd_ref[...],
                  preferred_element_type=jnp.float32) + b_ref[...]
    acc = acc * deg
    acc += jnp.dot(s0_ref[...] + s1_ref[...], ws_ref[...],
                   preferred_element_type=jnp.float32)
    acc += jnp.dot(p[:, 0:DE], we_ref[...],
                   preferred_element_type=jnp.float32)
    o_ref[...] = acc


_tc_call = pl.pallas_call(
    _tc_body,
    out_shape=jax.ShapeDtypeStruct((N, D), jnp.float32),
    grid=(GRID,),
    in_specs=[
        pl.BlockSpec((ROWS, D), lambda i: (i, 0)),
        pl.BlockSpec((ROWS, D), lambda i: (i, 0)),
        pl.BlockSpec((ROWS, D), lambda i: (i + GRID, 0)),
        pl.BlockSpec((ROWS, D), lambda i: (i, 0)),
        pl.BlockSpec((ROWS, D), lambda i: (i + GRID, 0)),
        pl.BlockSpec((D, D), lambda i: (0, 0)),
        pl.BlockSpec((D, D), lambda i: (0, 0)),
        pl.BlockSpec((DE, D), lambda i: (0, 0)),
        pl.BlockSpec((1, D), lambda i: (0, 0)),
    ],
    out_specs=pl.BlockSpec((ROWS, D), lambda i: (i, 0)),
)


def kernel(nfeat, efeat, edge_index, W, b):
    src = edge_index[0]
    dst = edge_index[1]
    z128 = jnp.zeros((CHUNK, D), jnp.float32)
    oh = jnp.zeros((CHUNK, 16), jnp.float32).at[:, 0].set(1.0)
    iota = jnp.arange(N, dtype=jnp.int32)
    ps, ptg = _get_sc_call()(nfeat, efeat, src, dst, z128, oh, iota)
    wd = W[:D]
    ws = W[D:2 * D]
    we = W[2 * D:]
    return _tc_call(nfeat, ps, ps, ptg, ptg,
                    wd, ws, we, b.reshape(1, D))
